# Initial kernel scaffold; baseline (speedup 1.0000x reference)
import jax
import jax.numpy as jnp
from jax.experimental import pallas as pl


def kernel(pred_id, target_id):
    raise NotImplementedError("write your pallas kernel here")


# speedup vs baseline: 34.0451x; 34.0451x over previous
import functools

import jax
import jax.numpy as jnp
from jax import lax
from jax.experimental import pallas as pl
from jax.experimental.pallas import tpu as pltpu

N = 160000
C = 256
O = 256
BLK = 2000
GRID = N // BLK


def _stage1(t_ref, x_ref, out_ref):
    i = pl.program_id(0)

    @pl.when(i == 0)
    def _init():
        out_ref[...] = jnp.zeros_like(out_ref)

    x = x_ref[...]
    t = t_ref[0]
    rs = jnp.sum(x, axis=1, keepdims=True)
    rq = jnp.sum(x * x, axis=1, keepdims=True)
    col = lax.broadcasted_iota(jnp.int32, (1, C), 1)
    oh = t == col
    zero = jnp.zeros((), jnp.float32)
    c1 = jnp.sum(jnp.where(oh, rs, zero), axis=0, keepdims=True)
    c2 = jnp.sum(jnp.where(oh, rq, zero), axis=0, keepdims=True)
    cc = jnp.sum(oh.astype(jnp.float32), axis=0, keepdims=True)
    out_ref[0:1, :] += c1
    out_ref[1:2, :] += c2
    out_ref[2:3, :] += cc

    @pl.when(i == GRID - 1)
    def _finalize():
        s1 = out_ref[0:1, :]
        s2 = out_ref[1:2, :]
        cnt = out_ref[2:3, :]
        valid = lax.broadcasted_iota(jnp.int32, (1, C), 1) < (O - 1)
        ne = cnt * float(C)
        ne_safe = jnp.maximum(ne, 1.0)
        mean = jnp.where(valid, s1 / ne_safe, 0.0)
        var = (s2 - ne * mean * mean) / jnp.maximum(ne - 1.0, 1.0)
        std = jnp.sqrt(jnp.maximum(var, 0.0))
        std = jnp.where(jnp.logical_and(cnt > 1.0, valid), std, 0.0)
        out_ref[3:4, :] = mean
        out_ref[4:5, :] = std


def _stage2(stats_ref, statsT_ref, stats_smem, out_ref, n_ref, a_ref):
    mean = stats_ref[3:4, :]
    stdr = stats_ref[4:5, :]
    cols = lax.broadcasted_iota(jnp.int32, (32, O), 1)

    def init_rb(rb, carry):
        mT = statsT_ref[pl.ds(rb * 32, 32), 3:4]
        d = mean - mT
        ad = jnp.abs(d)
        n = 16.0 * ad
        n_ref[pl.ds(rb * 32, 32), :] = n
        a_ref[pl.ds(rb * 32, 32), :] = ad / (n + 1e-5)
        return carry

    lax.fori_loop(0, 8, init_rb, 0)

    def outer(rb, tot):
        ab = a_ref[pl.ds(rb * 32, 32), :]
        eb = ab * stdr + (1.0 + 1e-5)

        def inner(c, acc):
            sc = stats_smem[4, c]
            nrow = n_ref[pl.ds(c, 1), :]
            return acc + (2.0 * nrow + 1e-5) / (eb + nrow + ab * sc)

        acc = lax.fori_loop(0, O, inner, jnp.zeros((32, O), jnp.float32))
        M = 1.0 - acc * (1.0 / float(C))
        rows = lax.broadcasted_iota(jnp.int32, (32, O), 0) + rb * 32
        sel = rows > cols
        val = -M * jnp.log(1.0 - M)
        return tot + jnp.sum(jnp.where(sel, val, 0.0))

    tot = lax.fori_loop(0, 8, outer, jnp.zeros((), jnp.float32))
    out_ref[0, 0] = tot * (2.0 / float(O * (O - 1)))


@jax.jit
def kernel(pred_id, target_id):
    t3 = target_id.astype(jnp.int32).reshape(GRID, BLK, 1)
    stats = pl.pallas_call(
        _stage1,
        grid=(GRID,),
        in_specs=[
            pl.BlockSpec((1, BLK, 1), lambda i: (i, 0, 0)),
            pl.BlockSpec((BLK, C), lambda i: (i, 0)),
        ],
        out_specs=pl.BlockSpec((8, C), lambda i: (0, 0)),
        out_shape=jax.ShapeDtypeStruct((8, C), jnp.float32),
    )(t3, pred_id)
    statsT = stats.T
    loss = pl.pallas_call(
        _stage2,
        in_specs=[
            pl.BlockSpec(memory_space=pltpu.VMEM),
            pl.BlockSpec(memory_space=pltpu.VMEM),
            pl.BlockSpec(memory_space=pltpu.SMEM),
        ],
        out_specs=pl.BlockSpec(memory_space=pltpu.SMEM),
        out_shape=jax.ShapeDtypeStruct((1, 1), jnp.float32),
        scratch_shapes=[
            pltpu.VMEM((O, O), jnp.float32),
            pltpu.VMEM((O, O), jnp.float32),
        ],
    )(stats, statsT, stats)
    return loss[0, 0]
